# baseline (speedup 1.0000x reference)
import jax
import jax.numpy as jnp
from jax.experimental import pallas as pl
from jax.experimental.pallas import tpu as pltpu

_GS = 64


def _norm_kernel(idx_ref, x_ref, o_ref, onehot_ref):
    n = onehot_ref.shape[0]
    ng = onehot_ref.shape[1]

    @pl.when(pl.program_id(0) == 0)
    def _build_onehot():
        gmat = (
            jax.lax.broadcasted_iota(jnp.int32, (n, ng), 0) // _GS
            == jax.lax.broadcasted_iota(jnp.int32, (n, ng), 1)
        ).astype(jnp.bfloat16)
        idx = idx_ref[...]
        cc = 512
        for ci in range(n // cc):
            cvals = ci * cc + jax.lax.broadcasted_iota(jnp.int32, (cc, n), 0)
            cmp = (cvals == idx).astype(jnp.bfloat16)
            onehot_ref[pl.ds(ci * cc, cc), :] = jax.lax.dot_general(
                cmp, gmat, (((1,), (0,)), ((), ())),
                preferred_element_type=jnp.float32,
            ).astype(jnp.bfloat16)

    x = x_ref[...]
    onehot = onehot_ref[...]
    denom = jax.lax.dot_general(
        x.astype(jnp.bfloat16), onehot, (((1,), (0,)), ((), ())),
        preferred_element_type=jnp.float32,
    )
    recip = (1.0 / denom).astype(jnp.bfloat16)
    rexp = jax.lax.dot_general(
        recip, onehot, (((1,), (1,)), ((), ())),
        preferred_element_type=jnp.float32,
    )
    o_ref[...] = x * rexp


@jax.jit
def kernel(x, index):
    b, n = x.shape
    ng = n // _GS
    rows = 128
    idx2 = index.reshape(1, n)
    return pl.pallas_call(
        _norm_kernel,
        grid=(b // rows,),
        in_specs=[
            pl.BlockSpec((1, n), lambda i: (0, 0)),
            pl.BlockSpec((rows, n), lambda i: (i, 0)),
        ],
        out_specs=pl.BlockSpec((rows, n), lambda i: (i, 0)),
        out_shape=jax.ShapeDtypeStruct((b, n), x.dtype),
        scratch_shapes=[pltpu.VMEM((n, ng), jnp.bfloat16)],
    )(idx2, x)


# speedup vs baseline: 1.1827x; 1.1827x over previous
import jax
import jax.numpy as jnp
from jax.experimental import pallas as pl
from jax.experimental.pallas import tpu as pltpu

_GS = 64


def _norm_kernel(idx_ref, x_ref, o_ref, onehot_ref):
    n = onehot_ref.shape[0]
    ng = onehot_ref.shape[1]

    @pl.when(pl.program_id(0) == 0)
    def _build_onehot():
        gmat = (
            jax.lax.broadcasted_iota(jnp.int32, (n, ng), 0) // _GS
            == jax.lax.broadcasted_iota(jnp.int32, (n, ng), 1)
        ).astype(jnp.bfloat16)
        idx = idx_ref[...]
        cc = 512
        for ci in range(n // cc):
            cvals = ci * cc + jax.lax.broadcasted_iota(jnp.int32, (cc, n), 0)
            cmp = (cvals == idx).astype(jnp.bfloat16)
            onehot_ref[pl.ds(ci * cc, cc), :] = jax.lax.dot_general(
                cmp, gmat, (((1,), (0,)), ((), ())),
                preferred_element_type=jnp.float32,
            ).astype(jnp.bfloat16)

    x = x_ref[...]
    onehot = onehot_ref[...]
    denom = jax.lax.dot_general(
        x.astype(jnp.bfloat16), onehot, (((1,), (0,)), ((), ())),
        preferred_element_type=jnp.float32,
    )
    recip = (1.0 / denom).astype(jnp.bfloat16)
    rexp = jax.lax.dot_general(
        recip, onehot, (((1,), (1,)), ((), ())),
        preferred_element_type=jnp.float32,
    )
    o_ref[...] = x * rexp


@jax.jit
def kernel(x, index):
    b, n = x.shape
    ng = n // _GS
    rows = 256
    idx2 = index.reshape(1, n)
    return pl.pallas_call(
        _norm_kernel,
        grid=(b // rows,),
        in_specs=[
            pl.BlockSpec((1, n), lambda i: (0, 0)),
            pl.BlockSpec((rows, n), lambda i: (i, 0)),
        ],
        out_specs=pl.BlockSpec((rows, n), lambda i: (i, 0)),
        out_shape=jax.ShapeDtypeStruct((b, n), x.dtype),
        scratch_shapes=[pltpu.VMEM((n, ng), jnp.bfloat16)],
    )(idx2, x)
